# Initial kernel scaffold; baseline (speedup 1.0000x reference)
import jax
import jax.numpy as jnp
from jax.experimental import pallas as pl


def kernel(x, edge_index, W1, b1, W2, b2, g1, be1, W3, b3, W4, b4, g2, be2, Wf1, bf1, Wf2, bf2):
    raise NotImplementedError("write your pallas kernel here")


# speedup vs baseline: 11.2116x; 11.2116x over previous
import functools

import jax
import jax.numpy as jnp
from jax import lax
from jax.experimental import pallas as pl
from jax.experimental.pallas import tpu as pltpu
from jax.experimental.pallas import tpu_sc as plsc

N = 10000
E = 320000
F = 128
D = 32
C = 16

NW = 32
CHUNK = 128
EPT = E // NW
NCH = -(-EPT // CHUNK)
EPT_PAD = NCH * CHUNK
NPAD = 10112
RPT = NPAD // 16


def _seg_kernel_body(y_hbm, src_hbm, dst_hbm, out_hbm,
                     src_v, dst_v, rows_a, rows_b, stage, acc, sem_a, sem_b, sem_i):
    c = lax.axis_index("c")
    s = lax.axis_index("s")
    w = c * 16 + s

    cp_src = pltpu.async_copy(src_hbm.at[w], src_v, sem_i)
    cp_dst = pltpu.async_copy(dst_hbm.at[w], dst_v, sem_i)

    zero = jnp.zeros((16,), jnp.float32)

    def zrow(i, _):
        stage[i, pl.ds(0, 16)] = zero
        stage[i, pl.ds(16, 16)] = zero
        return 0

    lax.fori_loop(0, RPT, zrow, 0)
    pltpu.sync_copy(stage, acc.at[pl.ds(s * RPT, RPT)])
    cp_src.wait()
    cp_dst.wait()
    plsc.subcore_barrier()

    first = pltpu.async_copy(y_hbm.at[src_v.at[0]], rows_a, sem_a)

    def step(j, _):
        even = (j % 2) == 0

        @pl.when(even)
        def _():
            @pl.when(j + 1 < NCH)
            def _():
                pltpu.async_copy(y_hbm.at[src_v.at[j + 1]], rows_b, sem_b)
            pltpu.make_async_copy(y_hbm.at[src_v.at[0]], rows_a, sem_a).wait()
            pltpu.sync_copy(rows_a, acc.at[dst_v.at[j]], add=True)

        @pl.when(jnp.logical_not(even))
        def _():
            @pl.when(j + 1 < NCH)
            def _():
                pltpu.async_copy(y_hbm.at[src_v.at[j + 1]], rows_a, sem_a)
            pltpu.make_async_copy(y_hbm.at[src_v.at[0]], rows_b, sem_b).wait()
            pltpu.sync_copy(rows_b, acc.at[dst_v.at[j]], add=True)

        return 0

    lax.fori_loop(0, NCH, step, 0)
    plsc.subcore_barrier()

    pltpu.sync_copy(acc.at[pl.ds(s * RPT, RPT)], stage)
    pltpu.sync_copy(stage, out_hbm.at[c, pl.ds(s * RPT, RPT)])


_seg_sum = pl.kernel(
    _seg_kernel_body,
    out_type=jax.ShapeDtypeStruct((2, NPAD, D), jnp.float32),
    mesh=plsc.VectorSubcoreMesh(core_axis_name="c", subcore_axis_name="s"),
    scratch_types=[
        pltpu.VMEM((NCH, CHUNK), jnp.int32),
        pltpu.VMEM((NCH, CHUNK), jnp.int32),
        pltpu.VMEM((CHUNK, D), jnp.float32),
        pltpu.VMEM((CHUNK, D), jnp.float32),
        pltpu.VMEM((RPT, D), jnp.float32),
        pltpu.VMEM_SHARED((NPAD, D), jnp.float32),
        pltpu.SemaphoreType.DMA,
        pltpu.SemaphoreType.DMA,
        pltpu.SemaphoreType.DMA,
    ],
    compiler_params=pltpu.CompilerParams(use_tc_tiling_on_sc=False),
)


def _mm_body(x_ref, w_ref, o_ref):
    o_ref[...] = jnp.dot(x_ref[...], w_ref[...],
                         preferred_element_type=jnp.float32)


_proj = pl.pallas_call(
    _mm_body,
    out_shape=jax.ShapeDtypeStruct((N, D), jnp.float32),
)


def _bn(h, g, b):
    m = jnp.mean(h, axis=0, keepdims=True)
    v = jnp.mean((h - m) ** 2, axis=0, keepdims=True)
    return (h - m) * jax.lax.rsqrt(v + 1e-5) * g + b


def _mid_body(y_ref, parts_ref, b1_ref, w2_ref, b2_ref, g1_ref, be1_ref,
              w3_ref, o_ref):
    agg = parts_ref[0, :N, :] + parts_ref[1, :N, :]
    z = jnp.maximum(y_ref[...] + agg + b1_ref[...], 0.0)
    h = jnp.dot(z, w2_ref[...], preferred_element_type=jnp.float32) + b2_ref[...]
    h = jnp.maximum(h, 0.0)
    h = _bn(h, g1_ref[...], be1_ref[...])
    o_ref[...] = jnp.dot(h, w3_ref[...], preferred_element_type=jnp.float32)


_mid = pl.pallas_call(
    _mid_body,
    out_shape=jax.ShapeDtypeStruct((N, D), jnp.float32),
)


def _tail_body(y_ref, parts_ref, b3_ref, w4_ref, b4_ref, g2_ref, be2_ref,
               wf1_ref, bf1_ref, wf2_ref, bf2_ref, o_ref):
    agg = parts_ref[0, :N, :] + parts_ref[1, :N, :]
    z = jnp.maximum(y_ref[...] + agg + b3_ref[...], 0.0)
    h = jnp.dot(z, w4_ref[...], preferred_element_type=jnp.float32) + b4_ref[...]
    h = jnp.maximum(h, 0.0)
    h = _bn(h, g2_ref[...], be2_ref[...])
    f = jnp.maximum(
        jnp.dot(h, wf1_ref[...], preferred_element_type=jnp.float32)
        + bf1_ref[...], 0.0)
    logits = jnp.dot(f, wf2_ref[...], preferred_element_type=jnp.float32) \
        + bf2_ref[...]
    mx = jnp.max(logits, axis=1, keepdims=True)
    sh = logits - mx
    lse = jnp.log(jnp.sum(jnp.exp(sh), axis=1, keepdims=True))
    o_ref[...] = sh - lse


_tail = pl.pallas_call(
    _tail_body,
    out_shape=jax.ShapeDtypeStruct((N, C), jnp.float32),
)


def kernel(x, edge_index, W1, b1, W2, b2, g1, be1, W3, b3, W4, b4, g2, be2,
           Wf1, bf1, Wf2, bf2):
    pad = NW * EPT_PAD - E
    src = jnp.concatenate([edge_index[0], jnp.zeros((pad,), jnp.int32)])
    dst = jnp.concatenate([edge_index[1], jnp.full((pad,), N, jnp.int32)])
    src = src.reshape(NW, NCH, CHUNK)
    dst = dst.reshape(NW, NCH, CHUNK)

    b1r, b2r, b3r, b4r = (v.reshape(1, D) for v in (b1, b2, b3, b4))
    g1r, be1r, g2r, be2r = (v.reshape(1, D) for v in (g1, be1, g2, be2))
    bf1r = bf1.reshape(1, D)
    bf2r = bf2.reshape(1, C)

    y1 = _proj(x, W1)
    parts1 = _seg_sum(y1, src, dst)
    y2 = _mid(y1, parts1, b1r, W2, b2r, g1r, be1r, W3)
    parts2 = _seg_sum(y2, src, dst)
    return _tail(y2, parts2, b3r, W4, b4r, g2r, be2r, Wf1, bf1r, Wf2, bf2r)
